# baseline (speedup 1.0000x reference)
import functools

import jax
import jax.numpy as jnp
from jax import lax
from jax.experimental import pallas as pl
from jax.experimental.pallas import tpu as pltpu
from jax.experimental.pallas import tpu_sc as plsc


def _argmin_body(x_ref, cb_ref, idx_ref):
    x = x_ref[...]
    cb = cb_ref[...]
    s = lax.dot_general(
        x, cb, (((1,), (1,)), ((), ())),
        preferred_element_type=jnp.float32,
    )
    x_sq = jnp.sum(x * x, axis=1, keepdims=True)
    c_sq = jnp.sum(cb * cb, axis=1)[None, :]
    d2 = x_sq - 2.0 * s + c_sq
    dist = jnp.sqrt(jnp.maximum(d2, 0.0))
    k = dist.shape[1]
    m = jnp.min(dist, axis=1, keepdims=True)
    iota = lax.broadcasted_iota(jnp.int32, dist.shape, 1)
    idx = jnp.min(jnp.where(dist == m, iota, k), axis=1)
    idx_ref[0, 0, :] = idx


def _tc_argmin(flat_x, codebook, block_n, block_off, nb):
    d = flat_x.shape[1]
    k = codebook.shape[0]
    out = pl.pallas_call(
        _argmin_body,
        grid=(nb,),
        in_specs=[
            pl.BlockSpec((block_n, d), lambda i: (i + block_off, 0)),
            pl.BlockSpec((k, d), lambda i: (0, 0)),
        ],
        out_specs=pl.BlockSpec((1, 1, block_n), lambda i: (i, 0, 0)),
        out_shape=jax.ShapeDtypeStruct((nb, 1, block_n), jnp.int32),
    )(flat_x, codebook)
    return out.reshape(nb * block_n)


def _sc_info():
    try:
        info = plsc.get_sparse_core_info()
        return info.num_cores, info.num_subcores
    except Exception:
        return 2, 16


def _make_sc_gather(n_part, d, part_off, chunk):
    nc, ns = _sc_info()
    nw = nc * ns
    b_per_w = n_part // nw
    assert b_per_w % chunk == 0 and chunk % 8 == 0 and chunk <= 128
    n_chunks = b_per_w // chunk
    mesh = plsc.VectorSubcoreMesh(core_axis_name="c", subcore_axis_name="s")

    @functools.partial(
        pl.kernel,
        mesh=mesh,
        scratch_types=(
            [pltpu.VMEM((b_per_w,), jnp.int32)]
            + [pltpu.VMEM((chunk, d), jnp.float32)] * n_chunks
            + [pltpu.SemaphoreType.DMA] * (2 * n_chunks)
        ),
    )
    def gather_kernel(table_hbm, idx_hbm, out_hbm, *refs):
        idx_v = refs[0]
        rows = refs[1:1 + n_chunks]
        gsem = refs[1 + n_chunks:1 + 2 * n_chunks]
        ssem = refs[1 + 2 * n_chunks:1 + 3 * n_chunks]
        wid = lax.axis_index("s") * nc + lax.axis_index("c")
        base = wid * b_per_w
        pltpu.sync_copy(idx_hbm.at[pl.ds(base, b_per_w)], idx_v)
        gaths = [
            pltpu.async_copy(
                table_hbm.at[idx_v.at[pl.ds(c * chunk, chunk)]],
                rows[c], gsem[c])
            for c in range(n_chunks)
        ]
        stores = []
        for c in range(n_chunks):
            gaths[c].wait()
            stores.append(pltpu.async_copy(
                rows[c],
                out_hbm.at[pl.ds(part_off + base + c * chunk, chunk)],
                ssem[c]))
        for st in stores:
            st.wait()

    return gather_kernel


def _alloc_body(o_ref):
    o_ref[...] = jnp.zeros_like(o_ref)


def _alloc_uninit(n, d):
    return pl.pallas_call(
        _alloc_body,
        grid=(1,),
        out_specs=pl.BlockSpec((8, 128), lambda i: (0, 0)),
        out_shape=jax.ShapeDtypeStruct((n, d), jnp.float32),
    )()


def kernel(x, codebook):
    d = x.shape[-1]
    flat_x = x.reshape(-1, d)
    n = flat_x.shape[0]
    block_n = 2048
    part_sizes = (10 * n // 16, 6 * n // 16)
    out_ref = jax.new_ref(_alloc_uninit(n, d))
    off = 0
    for np_ in part_sizes:
        idx = _tc_argmin(flat_x, codebook, block_n,
                         off // block_n, np_ // block_n)
        _make_sc_gather(np_, d, off, chunk=64)(codebook, idx, out_ref)
        off += np_
    return out_ref[...].reshape(x.shape)


# speedup vs baseline: 1.0848x; 1.0848x over previous
import functools

import jax
import jax.numpy as jnp
from jax import lax
from jax.experimental import pallas as pl
from jax.experimental.pallas import tpu as pltpu
from jax.experimental.pallas import tpu_sc as plsc


def _argmin_body(x_ref, cb_ref, idx_ref):
    x = x_ref[...]
    cb = cb_ref[...]
    s = lax.dot_general(
        x, cb, (((1,), (1,)), ((), ())),
        preferred_element_type=jnp.float32,
    )
    x_sq = jnp.sum(x * x, axis=1, keepdims=True)
    c_sq = jnp.sum(cb * cb, axis=1)[None, :]
    d2 = x_sq - 2.0 * s + c_sq
    dist = jnp.sqrt(jnp.maximum(d2, 0.0))
    k = dist.shape[1]
    m = jnp.min(dist, axis=1, keepdims=True)
    iota = lax.broadcasted_iota(jnp.int32, dist.shape, 1)
    idx = jnp.min(jnp.where(dist == m, iota, k), axis=1)
    idx_ref[0, 0, :] = idx


def _tc_argmin(flat_x, codebook, block_n, block_off, nb):
    d = flat_x.shape[1]
    k = codebook.shape[0]
    out = pl.pallas_call(
        _argmin_body,
        grid=(nb,),
        in_specs=[
            pl.BlockSpec((block_n, d), lambda i: (i + block_off, 0)),
            pl.BlockSpec((k, d), lambda i: (0, 0)),
        ],
        out_specs=pl.BlockSpec((1, 1, block_n), lambda i: (i, 0, 0)),
        out_shape=jax.ShapeDtypeStruct((nb, 1, block_n), jnp.int32),
    )(flat_x, codebook)
    return out.reshape(nb * block_n)


def _sc_info():
    try:
        info = plsc.get_sparse_core_info()
        return info.num_cores, info.num_subcores
    except Exception:
        return 2, 16


def _make_sc_gather(n_part, d, part_off, chunk):
    nc, ns = _sc_info()
    nw = nc * ns
    b_per_w = n_part // nw
    assert b_per_w % chunk == 0 and chunk % 8 == 0 and chunk <= 128
    n_chunks = b_per_w // chunk
    mesh = plsc.VectorSubcoreMesh(core_axis_name="c", subcore_axis_name="s")

    @functools.partial(
        pl.kernel,
        mesh=mesh,
        scratch_types=(
            [pltpu.VMEM((b_per_w,), jnp.int32)]
            + [pltpu.VMEM((chunk, d), jnp.float32)] * n_chunks
            + [pltpu.SemaphoreType.DMA] * (2 * n_chunks)
        ),
    )
    def gather_kernel(table_hbm, idx_hbm, out_hbm, *refs):
        idx_v = refs[0]
        rows = refs[1:1 + n_chunks]
        gsem = refs[1 + n_chunks:1 + 2 * n_chunks]
        ssem = refs[1 + 2 * n_chunks:1 + 3 * n_chunks]
        wid = lax.axis_index("s") * nc + lax.axis_index("c")
        base = wid * b_per_w
        pltpu.sync_copy(idx_hbm.at[pl.ds(base, b_per_w)], idx_v)
        gaths = [
            pltpu.async_copy(
                table_hbm.at[idx_v.at[pl.ds(c * chunk, chunk)]],
                rows[c], gsem[c])
            for c in range(n_chunks)
        ]
        stores = []
        for c in range(n_chunks):
            gaths[c].wait()
            stores.append(pltpu.async_copy(
                rows[c],
                out_hbm.at[pl.ds(part_off + base + c * chunk, chunk)],
                ssem[c]))
        for st in stores:
            st.wait()

    return gather_kernel


def _alloc_body(o_ref):
    o_ref[...] = jnp.zeros_like(o_ref)


def _alloc_uninit(n, d):
    return pl.pallas_call(
        _alloc_body,
        grid=(1,),
        out_specs=pl.BlockSpec((8, 128), lambda i: (0, 0)),
        out_shape=jax.ShapeDtypeStruct((n, d), jnp.float32),
    )()


def kernel(x, codebook):
    d = x.shape[-1]
    flat_x = x.reshape(-1, d)
    n = flat_x.shape[0]
    block_n = 2048
    part_sizes = (n // 2, n // 2)
    out_ref = jax.new_ref(_alloc_uninit(n, d))
    off = 0
    for np_ in part_sizes:
        idx = _tc_argmin(flat_x, codebook, block_n,
                         off // block_n, np_ // block_n)
        _make_sc_gather(np_, d, off, chunk=128)(codebook, idx, out_ref)
        off += np_
    return out_ref[...].reshape(x.shape)


# speedup vs baseline: 1.0923x; 1.0070x over previous
import functools

import jax
import jax.numpy as jnp
from jax import lax
from jax.experimental import pallas as pl
from jax.experimental.pallas import tpu as pltpu
from jax.experimental.pallas import tpu_sc as plsc


def _argmin_body(x_ref, cb_ref, idx_ref):
    x = x_ref[...]
    cb = cb_ref[...]
    s = lax.dot_general(
        x, cb, (((1,), (1,)), ((), ())),
        preferred_element_type=jnp.float32,
    )
    x_sq = jnp.sum(x * x, axis=1, keepdims=True)
    c_sq = jnp.sum(cb * cb, axis=1)[None, :]
    d2 = x_sq - 2.0 * s + c_sq
    dist = jnp.sqrt(jnp.maximum(d2, 0.0))
    k = dist.shape[1]
    m = jnp.min(dist, axis=1, keepdims=True)
    iota = lax.broadcasted_iota(jnp.int32, dist.shape, 1)
    idx = jnp.min(jnp.where(dist == m, iota, k), axis=1)
    idx_ref[0, 0, :] = idx


def _tc_argmin(flat_x, codebook, block_n, block_off, nb):
    d = flat_x.shape[1]
    k = codebook.shape[0]
    out = pl.pallas_call(
        _argmin_body,
        grid=(nb,),
        in_specs=[
            pl.BlockSpec((block_n, d), lambda i: (i + block_off, 0)),
            pl.BlockSpec((k, d), lambda i: (0, 0)),
        ],
        out_specs=pl.BlockSpec((1, 1, block_n), lambda i: (i, 0, 0)),
        out_shape=jax.ShapeDtypeStruct((nb, 1, block_n), jnp.int32),
    )(flat_x, codebook)
    return out.reshape(nb * block_n)


def _sc_info():
    try:
        info = plsc.get_sparse_core_info()
        return info.num_cores, info.num_subcores
    except Exception:
        return 2, 16


def _make_sc_gather(n_part, d, part_off, chunk):
    nc, ns = _sc_info()
    nw = nc * ns
    b_per_w = n_part // nw
    assert b_per_w % chunk == 0 and chunk % 8 == 0 and chunk <= 128
    n_chunks = b_per_w // chunk
    mesh = plsc.VectorSubcoreMesh(core_axis_name="c", subcore_axis_name="s")

    @functools.partial(
        pl.kernel,
        mesh=mesh,
        scratch_types=(
            [pltpu.VMEM((b_per_w,), jnp.int32)]
            + [pltpu.VMEM((chunk, d), jnp.float32)] * n_chunks
            + [pltpu.SemaphoreType.DMA] * (2 * n_chunks)
        ),
    )
    def gather_kernel(table_hbm, idx_hbm, out_hbm, *refs):
        idx_v = refs[0]
        rows = refs[1:1 + n_chunks]
        gsem = refs[1 + n_chunks:1 + 2 * n_chunks]
        ssem = refs[1 + 2 * n_chunks:1 + 3 * n_chunks]
        wid = lax.axis_index("s") * nc + lax.axis_index("c")
        base = wid * b_per_w
        pltpu.sync_copy(idx_hbm.at[pl.ds(base, b_per_w)], idx_v)
        gaths = [
            pltpu.async_copy(
                table_hbm.at[idx_v.at[pl.ds(c * chunk, chunk)]],
                rows[c], gsem[c])
            for c in range(n_chunks)
        ]
        stores = []
        for c in range(n_chunks):
            gaths[c].wait()
            stores.append(pltpu.async_copy(
                rows[c],
                out_hbm.at[pl.ds(part_off + base + c * chunk, chunk)],
                ssem[c]))
        for st in stores:
            st.wait()

    return gather_kernel


def _alloc_body(o_ref):
    o_ref[...] = jnp.zeros_like(o_ref)


def _alloc_uninit(n, d):
    return pl.pallas_call(
        _alloc_body,
        grid=(1,),
        out_specs=pl.BlockSpec((8, 128), lambda i: (0, 0)),
        out_shape=jax.ShapeDtypeStruct((n, d), jnp.float32),
    )()


def kernel(x, codebook):
    d = x.shape[-1]
    flat_x = x.reshape(-1, d)
    n = flat_x.shape[0]
    block_n = 4096
    part_sizes = (n // 2, n // 2)
    out_ref = jax.new_ref(_alloc_uninit(n, d))
    off = 0
    for np_ in part_sizes:
        idx = _tc_argmin(flat_x, codebook, block_n,
                         off // block_n, np_ // block_n)
        _make_sc_gather(np_, d, off, chunk=128)(codebook, idx, out_ref)
        off += np_
    return out_ref[...].reshape(x.shape)
